# baseline (speedup 1.0000x reference)
import functools

import jax
import jax.numpy as jnp
from jax import lax
from jax.experimental import pallas as pl
from jax.experimental.pallas import tpu as pltpu
from jax.experimental.pallas import tpu_sc as plsc

_C = 1000
_C_PAD = 1024
_SP_PAD = 1152
_ROWS = 320000
_SC_ROWS = 230400
_TC_ROWS = _ROWS - _SC_ROWS
_TC_BLK = 1600
_D = 128
_QB = 1024
_NC = 2
_NS = 16
_NW = _NC * _NS
_RPW = _SC_ROWS // _NW
_CH = 80
_NCH = _RPW // _CH
_GP = _CH // 16
_W = 128


def _rsqrt_s(x):
    i = lax.bitcast_convert_type(x, jnp.int32)
    i = jnp.int32(0x5F3759DF) - lax.shift_right_logical(i, 1)
    y = lax.bitcast_convert_type(i, jnp.float32)
    for _ in range(3):
        y = y * (jnp.float32(1.5) - jnp.float32(0.5) * x * y * y)
    return y


def _row_scale(buf, r):
    vs = [buf[r, pl.ds(jj * 16, 16)] for jj in range(8)]
    p0 = vs[0] * vs[0] + vs[1] * vs[1]
    p1 = vs[2] * vs[2] + vs[3] * vs[3]
    p2 = vs[4] * vs[4] + vs[5] * vs[5]
    p3 = vs[6] * vs[6] + vs[7] * vs[7]
    tot = jnp.sum((p0 + p1) + (p2 + p3))
    return vs, _rsqrt_s(jnp.maximum(tot, jnp.float32(1e-16)))


def _sc_proto_body(feats_hbm, labs_hbm, sums_out, cnt_out,
                   labs_v, b0, b1, win, cntw, idxb, sp_tab, sp_cnt, sem0, sem1):
    c = lax.axis_index("c")
    s = lax.axis_index("s")
    w = c * _NS + s
    row0 = w * _RPW
    iota = lax.iota(jnp.int32, 16)
    z16 = jnp.zeros((16,), jnp.float32)

    def zero_win():
        def zr(r, _):
            for jj in range(8):
                win[r, pl.ds(jj * 16, 16)] = z16
                cntw[r, pl.ds(jj * 16, 16)] = z16
            return 0
        lax.fori_loop(0, _W, zr, 0)

    zero_win()
    stripe = _SP_PAD // _NS
    pltpu.sync_copy(win.at[pl.ds(0, stripe)], sp_tab.at[pl.ds(s * stripe, stripe)])
    pltpu.sync_copy(cntw.at[pl.ds(0, stripe)], sp_cnt.at[pl.ds(s * stripe, stripe)])
    plsc.subcore_barrier()

    pltpu.sync_copy(labs_hbm.at[pl.ds(row0, _RPW)], labs_v.at[pl.ds(0, _RPW)])

    def flush(base):
        for jj in range(8):
            idxb[pl.ds(jj * 16, 16)] = base + iota + (jj * 16)
        pltpu.sync_copy(win, sp_tab.at[idxb], add=True)
        pltpu.sync_copy(cntw, sp_cnt.at[idxb], add=True)
        zero_win()

    def spill(base, cls, acc):
        loc = cls - base
        for jj in range(8):
            plsc.addupdate(win.at[loc, pl.ds(jj * 16, 16)], acc[jj])

    def group(rg, r0buf, buf, st):
        lv = labs_v[pl.ds(rg, 16)]

        def fast(st):
            base, cls, *acc = st

            def new_cls(args):
                cls, acc = args
                spill(base, cls, acc)
                return lv[0], [z16] * 8

            cls, acc = lax.cond(lv[0] != cls, new_cls,
                                lambda args: (args[0], list(args[1])), (cls, acc))
            plsc.addupdate_scatter(cntw, [lv - base, iota],
                                   jnp.ones((16,), jnp.float32))
            for k in range(16):
                vs, sck = _row_scale(buf, r0buf + k)
                for jj in range(8):
                    acc[jj] = acc[jj] + vs[jj] * sck
            return (base, cls, *acc)

        def slow(st):
            base, cls, *acc = st
            spill(base, cls, acc)
            scals = []
            for k in range(16):
                _, sck = _row_scale(buf, r0buf + k)
                scals.append(sck)

            def pass_cond(s2):
                return s2[1] < 16

            def pass_body(s2):
                base, nd = s2
                inwin = (lv < base + _W) & (iota >= nd)
                local = lv - base
                plsc.addupdate_scatter(cntw, [local, iota],
                                       jnp.ones((16,), jnp.float32), mask=inwin)
                ncov = nd + plsc.all_reduce_population_count(inwin)[0]
                one = jnp.float32(1.0)
                zero = jnp.float32(0.0)
                for k in range(16):
                    loc_k = jnp.clip(lv[k] - base, 0, _W - 1)
                    gate = jnp.where((nd <= k) & (ncov > k), one, zero)
                    sck = scals[k] * gate
                    for jj in range(8):
                        v = buf[r0buf + k, pl.ds(jj * 16, 16)]
                        plsc.addupdate(win.at[loc_k, pl.ds(jj * 16, 16)], v * sck)

                def do_flush(b):
                    flush(b)
                    return labs_v[pl.ds(rg + ncov, 16)][0]

                newbase = lax.cond(ncov < 16, do_flush, lambda b: b, base)
                return (newbase, ncov)

            base, _ = lax.while_loop(pass_cond, pass_body, (base, jnp.int32(0)))
            return (base, lv[15], *([z16] * 8))

        base = st[0]
        return lax.cond((lv[0] == lv[15]) & (lv[15] < base + _W), fast, slow, st)

    def start_dma(ci, bref, sem):
        pltpu.async_copy(feats_hbm.at[pl.ds(row0 + ci * _CH, _CH)], bref, sem)

    def wait_dma(ci, bref, sem):
        pltpu.make_async_copy(feats_hbm.at[pl.ds(row0 + ci * _CH, _CH)],
                              bref, sem).wait()

    def proc_chunk(ci, buf, st):
        def gb(g, s2):
            return group(ci * _CH + g * 16, g * 16, buf, s2)

        return lax.fori_loop(0, _GP, gb, st)

    start_dma(0, b0, sem0)

    def pair_body(p, st):
        ci = 2 * p
        wait_dma(ci, b0, sem0)
        start_dma(ci + 1, b1, sem1)
        st = proc_chunk(ci, b0, st)
        wait_dma(ci + 1, b1, sem1)
        start_dma(ci + 2, b0, sem0)
        return proc_chunk(ci + 1, b1, st)

    base0 = labs_v[pl.ds(0, 16)][0]
    st = lax.fori_loop(0, (_NCH - 1) // 2, pair_body, (base0, base0, *([z16] * 8)))
    wait_dma(_NCH - 1, b0, sem0)
    st = proc_chunk(_NCH - 1, b0, st)
    base, cls, *acc = st
    spill(base, cls, acc)
    flush(base)
    plsc.subcore_barrier()

    pltpu.sync_copy(sp_tab.at[pl.ds(s * 64, 64)], sums_out.at[c, pl.ds(s * 64, 64)])
    pltpu.sync_copy(sp_cnt.at[pl.ds(s * 64, 64)], cnt_out.at[c, pl.ds(s * 64, 64)])


def _sc_proto(support_feats, labels_i32):
    mesh = plsc.VectorSubcoreMesh(core_axis_name="c", subcore_axis_name="s")
    f = functools.partial(
        pl.kernel,
        mesh=mesh,
        compiler_params=pltpu.CompilerParams(needs_layout_passes=False),
        out_type=[
            jax.ShapeDtypeStruct((_NC, _C_PAD, _D), jnp.float32),
            jax.ShapeDtypeStruct((_NC, _C_PAD, _D), jnp.float32),
        ],
        scratch_types=[
            pltpu.VMEM((_RPW + 16,), jnp.int32),
            pltpu.VMEM((_CH, _D), jnp.float32),
            pltpu.VMEM((_CH, _D), jnp.float32),
            pltpu.VMEM((_W, _D), jnp.float32),
            pltpu.VMEM((_W, _D), jnp.float32),
            pltpu.VMEM((_W,), jnp.int32),
            pltpu.VMEM_SHARED((_SP_PAD, _D), jnp.float32),
            pltpu.VMEM_SHARED((_SP_PAD, _D), jnp.float32),
            pltpu.SemaphoreType.DMA,
            pltpu.SemaphoreType.DMA,
        ],
    )(_sc_proto_body)
    return f(support_feats, labels_i32)


def _proto_tc_kernel(lab_ref, sf_ref, sum_ref, cnt_ref):
    @pl.when(pl.program_id(0) == 0)
    def _init():
        sum_ref[...] = jnp.zeros_like(sum_ref)
        cnt_ref[...] = jnp.zeros_like(cnt_ref)

    sf = sf_ref[...]
    ssq = jnp.sum(sf * sf, axis=1, keepdims=True)
    sfn = sf * lax.rsqrt(jnp.maximum(ssq, 1e-16))
    lab = lab_ref[0, 0, :]
    oh = lab[:, None] == lax.broadcasted_iota(jnp.int32, (_TC_BLK, _C_PAD), 1)
    sum_ref[...] += lax.dot_general(
        oh.astype(jnp.bfloat16), sfn.astype(jnp.bfloat16),
        (((0,), (0,)), ((), ())), preferred_element_type=jnp.float32)
    cnt_ref[...] += jnp.broadcast_to(
        jnp.sum(oh, axis=0, dtype=jnp.float32)[None, :], (8, _C_PAD))


def _dist_kernel(nc_ref, cnt_ref, sum_ref, tcc_ref, tcs_ref, qf_ref, out_ref):
    qf = qf_ref[...]
    qn = qf * lax.rsqrt(jnp.maximum(jnp.sum(qf * qf, axis=1, keepdims=True), 1e-16))
    qsq = jnp.sum(qn * qn, axis=1, keepdims=True)
    sums = sum_ref[0] + sum_ref[1] + tcs_ref[...]
    cnt2 = cnt_ref[0] + cnt_ref[1]
    cnt = lax.dot_general(jnp.ones((8, _D), jnp.float32), cnt2,
                          (((1,), (1,)), ((), ())),
                          preferred_element_type=jnp.float32)[0:1, :] + tcc_ref[0:1, :]
    inv = 1.0 / jnp.maximum(cnt, 1.0)
    raw = lax.dot_general(qn, sums, (((1,), (1,)), ((), ())),
                          preferred_element_type=jnp.float32)
    s2 = lax.dot_general(jnp.ones((8, _D), jnp.float32), sums * sums,
                         (((1,), (1,)), ((), ())),
                         preferred_element_type=jnp.float32)[0:1, :]
    psq = s2 * inv * inv
    logits = 4.0 * raw * inv - 2.0 * qsq - 2.0 * psq
    col = lax.broadcasted_iota(jnp.int32, (1, _C_PAD), 1)
    present = (cnt > 0.0) & (col < nc_ref[0])
    out_ref[...] = jnp.where(present, logits, jnp.float32(-1e6))[:, :_C]


def kernel(support_feats, support_labels, query_feats, num_classes):
    sums, cnt = _sc_proto(support_feats, support_labels)

    noff = _SC_ROWS // _TC_BLK
    lab3 = support_labels.reshape(_ROWS // _TC_BLK, 1, _TC_BLK)
    tcs, tcc = pl.pallas_call(
        _proto_tc_kernel,
        grid=(_TC_ROWS // _TC_BLK,),
        in_specs=[
            pl.BlockSpec((1, 1, _TC_BLK), lambda i: (i + noff, 0, 0)),
            pl.BlockSpec((_TC_BLK, _D), lambda i: (i + noff, 0)),
        ],
        out_specs=[
            pl.BlockSpec((_C_PAD, _D), lambda i: (0, 0)),
            pl.BlockSpec((8, _C_PAD), lambda i: (0, 0)),
        ],
        out_shape=[
            jax.ShapeDtypeStruct((_C_PAD, _D), jnp.float32),
            jax.ShapeDtypeStruct((8, _C_PAD), jnp.float32),
        ],
    )(lab3, support_feats)

    nc = jnp.asarray(num_classes, jnp.int32).reshape(1)
    nq = query_feats.shape[0] // _QB
    out = pl.pallas_call(
        _dist_kernel,
        grid=(nq,),
        in_specs=[
            pl.BlockSpec(memory_space=pltpu.SMEM),
            pl.BlockSpec((_NC, _C_PAD, _D), lambda i: (0, 0, 0)),
            pl.BlockSpec((_NC, _C_PAD, _D), lambda i: (0, 0, 0)),
            pl.BlockSpec((8, _C_PAD), lambda i: (0, 0)),
            pl.BlockSpec((_C_PAD, _D), lambda i: (0, 0)),
            pl.BlockSpec((_QB, _D), lambda i: (i, 0)),
        ],
        out_specs=pl.BlockSpec((_QB, _C), lambda i: (i, 0)),
        out_shape=jax.ShapeDtypeStruct((query_feats.shape[0], _C), jnp.float32),
    )(nc, cnt, sums, tcc, tcs, query_feats)
    return out


# speedup vs baseline: 1.0647x; 1.0647x over previous
import functools

import jax
import jax.numpy as jnp
from jax import lax
from jax.experimental import pallas as pl
from jax.experimental.pallas import tpu as pltpu
from jax.experimental.pallas import tpu_sc as plsc

_C = 1000
_C_PAD = 1024
_SP_PAD = 1152
_ROWS = 320000
_SC_ROWS = 217600
_TC_ROWS = _ROWS - _SC_ROWS
_TC_BLK = 1600
_D = 128
_QB = 1024
_NC = 2
_NS = 16
_NW = _NC * _NS
_RPW = _SC_ROWS // _NW
_CH = 80
_NCH = _RPW // _CH
_GP = _CH // 16
_W = 128


def _rsqrt_s(x):
    i = lax.bitcast_convert_type(x, jnp.int32)
    i = jnp.int32(0x5F3759DF) - lax.shift_right_logical(i, 1)
    y = lax.bitcast_convert_type(i, jnp.float32)
    for _ in range(2):
        y = y * (jnp.float32(1.5) - jnp.float32(0.5) * x * y * y)
    return y


def _row_scale(buf, r):
    vs = [buf[r, pl.ds(jj * 16, 16)] for jj in range(8)]
    p0 = vs[0] * vs[0] + vs[1] * vs[1]
    p1 = vs[2] * vs[2] + vs[3] * vs[3]
    p2 = vs[4] * vs[4] + vs[5] * vs[5]
    p3 = vs[6] * vs[6] + vs[7] * vs[7]
    tot = jnp.sum((p0 + p1) + (p2 + p3))
    return vs, _rsqrt_s(jnp.maximum(tot, jnp.float32(1e-16)))


def _sc_proto_body(feats_hbm, labs_hbm, sums_out, cnt_out,
                   labs_v, b0, b1, win, cntw, idxb, sp_tab, sp_cnt, sem0, sem1):
    c = lax.axis_index("c")
    s = lax.axis_index("s")
    w = c * _NS + s
    row0 = w * _RPW
    iota = lax.iota(jnp.int32, 16)
    z16 = jnp.zeros((16,), jnp.float32)

    def zero_win():
        def zr(r, _):
            for jj in range(8):
                win[r, pl.ds(jj * 16, 16)] = z16
                cntw[r, pl.ds(jj * 16, 16)] = z16
            return 0
        lax.fori_loop(0, _W, zr, 0)

    zero_win()
    stripe = _SP_PAD // _NS
    pltpu.sync_copy(win.at[pl.ds(0, stripe)], sp_tab.at[pl.ds(s * stripe, stripe)])
    pltpu.sync_copy(cntw.at[pl.ds(0, stripe)], sp_cnt.at[pl.ds(s * stripe, stripe)])
    plsc.subcore_barrier()

    pltpu.sync_copy(labs_hbm.at[pl.ds(row0, _RPW)], labs_v.at[pl.ds(0, _RPW)])

    def flush(base):
        for jj in range(8):
            idxb[pl.ds(jj * 16, 16)] = base + iota + (jj * 16)
        pltpu.sync_copy(win, sp_tab.at[idxb], add=True)
        pltpu.sync_copy(cntw, sp_cnt.at[idxb], add=True)
        zero_win()

    def spill(base, cls, acc):
        loc = cls - base
        for jj in range(8):
            plsc.addupdate(win.at[loc, pl.ds(jj * 16, 16)], acc[jj])

    def group(rg, r0buf, buf, st):
        lv = labs_v[pl.ds(rg, 16)]

        def fast(st):
            base, cls, *acc = st

            def new_cls(args):
                cls, acc = args
                spill(base, cls, acc)
                return lv[0], [z16] * 8

            cls, acc = lax.cond(lv[0] != cls, new_cls,
                                lambda args: (args[0], list(args[1])), (cls, acc))
            plsc.addupdate_scatter(cntw, [lv - base, iota],
                                   jnp.ones((16,), jnp.float32))
            for k in range(16):
                vs, sck = _row_scale(buf, r0buf + k)
                for jj in range(8):
                    acc[jj] = acc[jj] + vs[jj] * sck
            return (base, cls, *acc)

        def slow(st):
            base, cls, *acc = st
            spill(base, cls, acc)
            scals = []
            for k in range(16):
                _, sck = _row_scale(buf, r0buf + k)
                scals.append(sck)

            def pass_cond(s2):
                return s2[1] < 16

            def pass_body(s2):
                base, nd = s2
                inwin = (lv < base + _W) & (iota >= nd)
                local = lv - base
                plsc.addupdate_scatter(cntw, [local, iota],
                                       jnp.ones((16,), jnp.float32), mask=inwin)
                ncov = nd + plsc.all_reduce_population_count(inwin)[0]
                one = jnp.float32(1.0)
                zero = jnp.float32(0.0)
                for k in range(16):
                    loc_k = jnp.clip(lv[k] - base, 0, _W - 1)
                    gate = jnp.where((nd <= k) & (ncov > k), one, zero)
                    sck = scals[k] * gate
                    for jj in range(8):
                        v = buf[r0buf + k, pl.ds(jj * 16, 16)]
                        plsc.addupdate(win.at[loc_k, pl.ds(jj * 16, 16)], v * sck)

                def do_flush(b):
                    flush(b)
                    return labs_v[pl.ds(rg + ncov, 16)][0]

                newbase = lax.cond(ncov < 16, do_flush, lambda b: b, base)
                return (newbase, ncov)

            base, _ = lax.while_loop(pass_cond, pass_body, (base, jnp.int32(0)))
            return (base, lv[15], *([z16] * 8))

        base = st[0]
        return lax.cond((lv[0] == lv[15]) & (lv[15] < base + _W), fast, slow, st)

    def start_dma(ci, bref, sem):
        pltpu.async_copy(feats_hbm.at[pl.ds(row0 + ci * _CH, _CH)], bref, sem)

    def wait_dma(ci, bref, sem):
        pltpu.make_async_copy(feats_hbm.at[pl.ds(row0 + ci * _CH, _CH)],
                              bref, sem).wait()

    def proc_chunk(ci, buf, st):
        def gb(g, s2):
            return group(ci * _CH + g * 16, g * 16, buf, s2)

        return lax.fori_loop(0, _GP, gb, st)

    start_dma(0, b0, sem0)

    def pair_body(p, st):
        ci = 2 * p
        wait_dma(ci, b0, sem0)
        start_dma(ci + 1, b1, sem1)
        st = proc_chunk(ci, b0, st)
        wait_dma(ci + 1, b1, sem1)
        start_dma(ci + 2, b0, sem0)
        return proc_chunk(ci + 1, b1, st)

    base0 = labs_v[pl.ds(0, 16)][0]
    st = lax.fori_loop(0, (_NCH - 1) // 2, pair_body, (base0, base0, *([z16] * 8)))
    wait_dma(_NCH - 1, b0, sem0)
    st = proc_chunk(_NCH - 1, b0, st)
    base, cls, *acc = st
    spill(base, cls, acc)
    flush(base)
    plsc.subcore_barrier()

    pltpu.sync_copy(sp_tab.at[pl.ds(s * 64, 64)], sums_out.at[c, pl.ds(s * 64, 64)])
    pltpu.sync_copy(sp_cnt.at[pl.ds(s * 64, 64)], cnt_out.at[c, pl.ds(s * 64, 64)])


def _sc_proto(support_feats, labels_i32):
    mesh = plsc.VectorSubcoreMesh(core_axis_name="c", subcore_axis_name="s")
    f = functools.partial(
        pl.kernel,
        mesh=mesh,
        compiler_params=pltpu.CompilerParams(needs_layout_passes=False),
        out_type=[
            jax.ShapeDtypeStruct((_NC, _C_PAD, _D), jnp.float32),
            jax.ShapeDtypeStruct((_NC, _C_PAD, _D), jnp.float32),
        ],
        scratch_types=[
            pltpu.VMEM((_RPW + 16,), jnp.int32),
            pltpu.VMEM((_CH, _D), jnp.float32),
            pltpu.VMEM((_CH, _D), jnp.float32),
            pltpu.VMEM((_W, _D), jnp.float32),
            pltpu.VMEM((_W, _D), jnp.float32),
            pltpu.VMEM((_W,), jnp.int32),
            pltpu.VMEM_SHARED((_SP_PAD, _D), jnp.float32),
            pltpu.VMEM_SHARED((_SP_PAD, _D), jnp.float32),
            pltpu.SemaphoreType.DMA,
            pltpu.SemaphoreType.DMA,
        ],
    )(_sc_proto_body)
    return f(support_feats, labels_i32)


def _proto_tc_kernel(lab_ref, sf_ref, sum_ref, cnt_ref):
    @pl.when(pl.program_id(0) == 0)
    def _init():
        sum_ref[...] = jnp.zeros_like(sum_ref)
        cnt_ref[...] = jnp.zeros_like(cnt_ref)

    sf = sf_ref[...]
    ssq = jnp.sum(sf * sf, axis=1, keepdims=True)
    sfn = sf * lax.rsqrt(jnp.maximum(ssq, 1e-16))
    lab = lab_ref[0, 0, :]
    oh = lab[:, None] == lax.broadcasted_iota(jnp.int32, (_TC_BLK, _C_PAD), 1)
    sum_ref[...] += lax.dot_general(
        oh.astype(jnp.bfloat16), sfn.astype(jnp.bfloat16),
        (((0,), (0,)), ((), ())), preferred_element_type=jnp.float32)
    cnt_ref[...] += jnp.broadcast_to(
        jnp.sum(oh, axis=0, dtype=jnp.float32)[None, :], (8, _C_PAD))


def _dist_kernel(nc_ref, cnt_ref, sum_ref, tcc_ref, tcs_ref, qf_ref, out_ref):
    qf = qf_ref[...]
    qn = qf * lax.rsqrt(jnp.maximum(jnp.sum(qf * qf, axis=1, keepdims=True), 1e-16))
    qsq = jnp.sum(qn * qn, axis=1, keepdims=True)
    sums = sum_ref[0] + sum_ref[1] + tcs_ref[...]
    cnt2 = cnt_ref[0] + cnt_ref[1]
    cnt = lax.dot_general(jnp.ones((8, _D), jnp.float32), cnt2,
                          (((1,), (1,)), ((), ())),
                          preferred_element_type=jnp.float32)[0:1, :] + tcc_ref[0:1, :]
    inv = 1.0 / jnp.maximum(cnt, 1.0)
    raw = lax.dot_general(qn, sums, (((1,), (1,)), ((), ())),
                          preferred_element_type=jnp.float32)
    s2 = lax.dot_general(jnp.ones((8, _D), jnp.float32), sums * sums,
                         (((1,), (1,)), ((), ())),
                         preferred_element_type=jnp.float32)[0:1, :]
    psq = s2 * inv * inv
    logits = 4.0 * raw * inv - 2.0 * qsq - 2.0 * psq
    col = lax.broadcasted_iota(jnp.int32, (1, _C_PAD), 1)
    present = (cnt > 0.0) & (col < nc_ref[0])
    out_ref[...] = jnp.where(present, logits, jnp.float32(-1e6))[:, :_C]


def kernel(support_feats, support_labels, query_feats, num_classes):
    sums, cnt = _sc_proto(support_feats, support_labels)

    noff = _SC_ROWS // _TC_BLK
    lab3 = support_labels.reshape(_ROWS // _TC_BLK, 1, _TC_BLK)
    tcs, tcc = pl.pallas_call(
        _proto_tc_kernel,
        grid=(_TC_ROWS // _TC_BLK,),
        in_specs=[
            pl.BlockSpec((1, 1, _TC_BLK), lambda i: (i + noff, 0, 0)),
            pl.BlockSpec((_TC_BLK, _D), lambda i: (i + noff, 0)),
        ],
        out_specs=[
            pl.BlockSpec((_C_PAD, _D), lambda i: (0, 0)),
            pl.BlockSpec((8, _C_PAD), lambda i: (0, 0)),
        ],
        out_shape=[
            jax.ShapeDtypeStruct((_C_PAD, _D), jnp.float32),
            jax.ShapeDtypeStruct((8, _C_PAD), jnp.float32),
        ],
    )(lab3, support_feats)

    nc = jnp.asarray(num_classes, jnp.int32).reshape(1)
    nq = query_feats.shape[0] // _QB
    out = pl.pallas_call(
        _dist_kernel,
        grid=(nq,),
        in_specs=[
            pl.BlockSpec(memory_space=pltpu.SMEM),
            pl.BlockSpec((_NC, _C_PAD, _D), lambda i: (0, 0, 0)),
            pl.BlockSpec((_NC, _C_PAD, _D), lambda i: (0, 0, 0)),
            pl.BlockSpec((8, _C_PAD), lambda i: (0, 0)),
            pl.BlockSpec((_C_PAD, _D), lambda i: (0, 0)),
            pl.BlockSpec((_QB, _D), lambda i: (i, 0)),
        ],
        out_specs=pl.BlockSpec((_QB, _C), lambda i: (i, 0)),
        out_shape=jax.ShapeDtypeStruct((query_feats.shape[0], _C), jnp.float32),
    )(nc, cnt, sums, tcc, tcs, query_feats)
    return out
